# baseline (speedup 1.0000x reference)
import jax
import jax.numpy as jnp
from jax.experimental import pallas as pl
from jax.experimental.pallas import tpu as pltpu

_CH = 80_000
_NCH = 25


def _body(x_hbm, out_ref, buf, sems):
    t = pl.program_id(0)

    def _cp(idx):
        return pltpu.make_async_copy(
            x_hbm.at[0, pl.ds(0, 3), pl.ds(idx * _CH, _CH)],
            buf.at[idx % 8],
            sems.at[idx % 8],
        )

    @pl.when(t == 0)
    def _init():
        out_ref[0, 0] = 0
        for k in range(7):
            _cp(k).start()

    @pl.when(t + 7 < _NCH)
    def _prefetch():
        _cp(t + 7).start()

    _cp(t).wait()
    nz = jnp.any(buf[t % 8, 1:3] != 0.0).astype(jnp.int32)
    out_ref[0, 0] = out_ref[0, 0] | nz


def kernel(x):
    xt = jnp.swapaxes(x, 0, 1)
    res = pl.pallas_call(
        _body,
        grid=(_NCH,),
        in_specs=[pl.BlockSpec(memory_space=pl.ANY)],
        out_specs=pl.BlockSpec(memory_space=pltpu.SMEM),
        out_shape=jax.ShapeDtypeStruct((1, 1), jnp.int32),
        scratch_shapes=[
            pltpu.VMEM((8, 3, _CH), jnp.float32),
            pltpu.SemaphoreType.DMA((8,)),
        ],
    )(xt)
    return (res[0, 0] != 0).reshape(1)


# speedup vs baseline: 1.1975x; 1.1975x over previous
import jax
import jax.numpy as jnp
from jax.experimental import pallas as pl
from jax.experimental.pallas import tpu as pltpu

_CH = 400_000
_NCH = 5


def _body(x_hbm, out_ref, buf, sems):
    t = pl.program_id(0)

    def _cp(idx):
        return pltpu.make_async_copy(
            x_hbm.at[0, pl.ds(1, 2), pl.ds(idx * _CH, _CH)],
            buf.at[idx % 2],
            sems.at[idx % 2],
        )

    @pl.when(t == 0)
    def _init():
        out_ref[0, 0] = 0
        for k in range(1):
            _cp(k).start()

    @pl.when(t + 1 < _NCH)
    def _prefetch():
        _cp(t + 1).start()

    _cp(t).wait()
    nz = jnp.any(buf[t % 2] != 0.0).astype(jnp.int32)
    out_ref[0, 0] = out_ref[0, 0] | nz


def kernel(x):
    xt = jnp.swapaxes(x, 0, 1)
    res = pl.pallas_call(
        _body,
        grid=(_NCH,),
        in_specs=[pl.BlockSpec(memory_space=pl.ANY)],
        out_specs=pl.BlockSpec(memory_space=pltpu.SMEM),
        out_shape=jax.ShapeDtypeStruct((1, 1), jnp.int32),
        scratch_shapes=[
            pltpu.VMEM((2, 2, _CH), jnp.float32),
            pltpu.SemaphoreType.DMA((2,)),
        ],
    )(xt)
    return (res[0, 0] != 0).reshape(1)
